# baseline (speedup 1.0000x reference)
import functools

import jax
import jax.numpy as jnp
from jax import lax
from jax.experimental import pallas as pl
from jax.experimental.pallas import tpu as pltpu
from jax.experimental.pallas import tpu_sc as plsc

NC = 2
NS = 16
NW = NC * NS
L = 16
BEDGE = 128
BM = 1000


def _mesh():
    return plsc.VectorSubcoreMesh(core_axis_name="c", subcore_axis_name="s")


_SC_PARAMS = pltpu.CompilerParams(needs_layout_passes=False)


def _sc_deg_body(ech, npad, row_h, col_h, w_h, z_h, out_h,
                 row_v, col_v, w_v, deg_v):
    wid = lax.axis_index("s") * NC + lax.axis_index("c")
    pltpu.sync_copy(z_h, deg_v)
    off = wid * ech
    pltpu.sync_copy(row_h.at[pl.ds(off, ech)], row_v)
    pltpu.sync_copy(col_h.at[pl.ds(off, ech)], col_v)
    pltpu.sync_copy(w_h.at[pl.ds(off, ech)], w_v)

    def body(i, carry):
        sl = pl.ds(i * L, L)
        r = row_v[sl]
        c = col_v[sl]
        w = w_v[sl]
        wz = jnp.where(r == c, 0.0, w)
        plsc.addupdate_scatter(deg_v, [r], wz)
        return carry

    lax.fori_loop(0, ech // L, body, 0)
    pltpu.sync_copy(deg_v, out_h.at[wid])


def _sc_deg(row, col, w, npad):
    e = row.shape[0]
    ech = e // NW
    z = jnp.zeros((npad,), jnp.float32)
    fn = pl.kernel(
        functools.partial(_sc_deg_body, ech, npad),
        out_type=jax.ShapeDtypeStruct((NW, npad), jnp.float32),
        mesh=_mesh(),
        compiler_params=_SC_PARAMS,
        scratch_types=[
            pltpu.VMEM((ech,), jnp.int32),
            pltpu.VMEM((ech,), jnp.int32),
            pltpu.VMEM((ech,), jnp.float32),
            pltpu.VMEM((npad,), jnp.float32),
        ],
    )
    return fn(row, col, w, z)


def _tc_dinv_body(d_ref, o_ref):
    d = jnp.sum(d_ref[...], axis=0)
    safe = jnp.where(d > 0.0, d, 1.0)
    o_ref[...] = jnp.where(d > 0.0, lax.rsqrt(safe), 0.0)


def _tc_dinv(degp, npad):
    rows = npad // 128
    degp3 = degp.reshape(NW, rows, 128)
    out = pl.pallas_call(
        _tc_dinv_body,
        grid=(rows // 8,),
        in_specs=[pl.BlockSpec((NW, 8, 128), lambda i: (0, i, 0))],
        out_specs=pl.BlockSpec((8, 128), lambda i: (i, 0)),
        out_shape=jax.ShapeDtypeStruct((rows, 128), jnp.float32),
    )(degp3)
    return out.reshape(npad)


def _sc_norm_body(ech, npad, row_h, col_h, w_h, dinv_h, out_h,
                  row_v, col_v, w_v, dv, nrm_v):
    wid = lax.axis_index("s") * NC + lax.axis_index("c")
    off = wid * ech
    pltpu.sync_copy(dinv_h, dv)
    pltpu.sync_copy(row_h.at[pl.ds(off, ech)], row_v)
    pltpu.sync_copy(col_h.at[pl.ds(off, ech)], col_v)
    pltpu.sync_copy(w_h.at[pl.ds(off, ech)], w_v)

    def body(i, carry):
        sl = pl.ds(i * L, L)
        r = row_v[sl]
        c = col_v[sl]
        w = w_v[sl]
        dr = plsc.load_gather(dv, [r])
        dc = plsc.load_gather(dv, [c])
        wz = jnp.where(r == c, 0.0, w)
        nrm_v[sl] = -(dr * wz * dc)
        return carry

    lax.fori_loop(0, ech // L, body, 0)
    pltpu.sync_copy(nrm_v, out_h.at[pl.ds(off, ech)])


def _sc_norm(row, col, w, dinv, npad):
    e = row.shape[0]
    ech = e // NW
    fn = pl.kernel(
        functools.partial(_sc_norm_body, ech, npad),
        out_type=jax.ShapeDtypeStruct((e,), jnp.float32),
        mesh=_mesh(),
        compiler_params=_SC_PARAMS,
        scratch_types=[
            pltpu.VMEM((ech,), jnp.int32),
            pltpu.VMEM((ech,), jnp.int32),
            pltpu.VMEM((ech,), jnp.float32),
            pltpu.VMEM((npad,), jnp.float32),
            pltpu.VMEM((ech,), jnp.float32),
        ],
    )
    return fn(row, col, w, dinv)


def _sc_prop_body(n, ech, d, row_h, col_h, nrm_h, h_h, z_h, out_h,
                  rows_v, rowb0, rowb1, rowb2,
                  colb0, colb1, colb2, colb3, colb4, colb5,
                  nrmb0, nrmb1, nrmb2,
                  sg0, sg1, sg2, ss0, ss1, ss2, scr0, scr1, scr2,
                  scc0, scc1, scc2, scc3, scc4, scc5,
                  scn0, scn1, scn2, acc_sp):
    cid = lax.axis_index("c")
    sid = lax.axis_index("s")
    wid = sid * NC + cid
    nblk = ech // BEDGE
    rowb = (rowb0, rowb1, rowb2)
    colb = (colb0, colb1, colb2, colb3, colb4, colb5)
    nrmb = (nrmb0, nrmb1, nrmb2)
    sg = (sg0, sg1, sg2)
    ss = (ss0, ss1, ss2)
    scr = (scr0, scr1, scr2)
    scc = (scc0, scc1, scc2, scc3, scc4, scc5)
    scn = (scn0, scn1, scn2)
    ebase = wid * ech
    rpta = (n // NS + 7) // 8 * 8
    rptz = n - (NS - 1) * rpta

    def idx_dma(i, s3, s6):
        off = ebase + i * BEDGE
        pltpu.async_copy(row_h.at[pl.ds(off, BEDGE)], rowb[s3], scr[s3])
        pltpu.async_copy(col_h.at[pl.ds(off, BEDGE)], colb[s6], scc[s6])
        pltpu.async_copy(nrm_h.at[pl.ds(off, BEDGE)], nrmb[s3], scn[s3])

    def idx_wait(s3, s6):
        pltpu.make_async_copy(row_h.at[pl.ds(0, BEDGE)], rowb[s3],
                              scr[s3]).wait()
        pltpu.make_async_copy(col_h.at[pl.ds(0, BEDGE)], colb[s6],
                              scc[s6]).wait()
        pltpu.make_async_copy(nrm_h.at[pl.ds(0, BEDGE)], nrmb[s3],
                              scn[s3]).wait()

    @pl.when(sid < NS - 1)
    def _():
        pltpu.sync_copy(z_h, acc_sp.at[pl.ds(sid * rpta, rpta)])

    @pl.when(sid == NS - 1)
    def _():
        pltpu.sync_copy(z_h.at[pl.ds(0, rptz)],
                        acc_sp.at[pl.ds((NS - 1) * rpta, rptz)])

    idx_dma(0, 0, 0)
    idx_dma(1, 1, 1)
    idx_wait(0, 0)
    pltpu.async_copy(h_h.at[rowb[0]], rows_v.at[0], sg[0])
    plsc.subcore_barrier()

    nsup = (nblk + 5) // 6

    def sup(s, carry):
        for b in range(6):
            b3 = b % 3
            nb3 = (b + 1) % 3
            i = s * 6 + b

            @pl.when(jnp.logical_and(i >= 2, i - 2 < nblk))
            def _():
                pltpu.make_async_copy(
                    rows_v.at[nb3], acc_sp.at[colb[0]], ss[nb3]).wait()

            @pl.when(i + 2 < nblk)
            def _():
                idx_dma(i + 2, (b + 2) % 3, (b + 2) % 6)

            @pl.when(i + 1 < nblk)
            def _():
                idx_wait(nb3, (b + 1) % 6)
                pltpu.async_copy(h_h.at[rowb[nb3]], rows_v.at[nb3], sg[nb3])

            @pl.when(i < nblk)
            def _():
                pltpu.make_async_copy(h_h.at[rowb[b3]], rows_v.at[b3],
                                      sg[b3]).wait()

                def edge(e2, c2):
                    s16 = plsc.load_gather(
                        nrmb[b3], [jnp.zeros((L,), jnp.int32) + e2])
                    for j in range(d // L):
                        sl = pl.ds(j * L, L)
                        rows_v[b3, e2, sl] = rows_v[b3, e2, sl] * s16
                    return c2

                lax.fori_loop(0, BEDGE, edge, 0, unroll=8)
                pltpu.async_copy(rows_v.at[b3], acc_sp.at[colb[b % 6]],
                                 ss[b3], add=True)
        return carry

    lax.fori_loop(0, nsup, sup, 0)
    for j in range(max(0, 6 * nsup - 2), nblk):
        pltpu.make_async_copy(rows_v.at[j % 3], acc_sp.at[colb[0]],
                              ss[j % 3]).wait()
    plsc.subcore_barrier()

    @pl.when(sid < NS - 1)
    def _():
        pltpu.sync_copy(acc_sp.at[pl.ds(sid * rpta, rpta)],
                        out_h.at[pl.ds(cid * n + sid * rpta, rpta)])

    @pl.when(sid == NS - 1)
    def _():
        pltpu.sync_copy(
            acc_sp.at[pl.ds((NS - 1) * rpta, rptz)],
            out_h.at[pl.ds(cid * n + (NS - 1) * rpta, rptz)])


def _sc_prop(h, rowp, colp, nrmp):
    n, d = h.shape
    e = rowp.shape[0]
    ech = e // NW
    rpta = (n // NS + 7) // 8 * 8
    z = jnp.zeros((rpta, d), jnp.float32)
    fn = pl.kernel(
        functools.partial(_sc_prop_body, n, ech, d),
        out_type=jax.ShapeDtypeStruct((NC * n, d), jnp.float32),
        mesh=_mesh(),
        compiler_params=_SC_PARAMS,
        scratch_types=[
            pltpu.VMEM((3, BEDGE, d), jnp.float32),
            pltpu.VMEM((BEDGE,), jnp.int32),
            pltpu.VMEM((BEDGE,), jnp.int32),
            pltpu.VMEM((BEDGE,), jnp.int32),
            pltpu.VMEM((BEDGE,), jnp.int32),
            pltpu.VMEM((BEDGE,), jnp.int32),
            pltpu.VMEM((BEDGE,), jnp.int32),
            pltpu.VMEM((BEDGE,), jnp.int32),
            pltpu.VMEM((BEDGE,), jnp.int32),
            pltpu.VMEM((BEDGE,), jnp.int32),
            pltpu.VMEM((BEDGE,), jnp.float32),
            pltpu.VMEM((BEDGE,), jnp.float32),
            pltpu.VMEM((BEDGE,), jnp.float32),
        ] + [pltpu.SemaphoreType.DMA] * 18 + [
            pltpu.VMEM_SHARED((n, d), jnp.float32),
        ],
    )
    return fn(rowp, colp, nrmp, h, z)


def _tc_comb_body(u_ref, p0_ref, p1_ref, w_ref, t1_ref, acc_ref):
    t1 = p0_ref[...] + p1_ref[...]
    t1_ref[...] = t1
    acc_ref[...] = (
        jnp.dot(u_ref[...], w_ref[0], preferred_element_type=jnp.float32)
        + jnp.dot(t1, w_ref[1], preferred_element_type=jnp.float32))


def _tc_comb(u, p0, p1, w):
    n, d = u.shape
    k = w.shape[0]
    grid = (n // BM,)
    blk = pl.BlockSpec((BM, d), lambda i: (i, 0))
    t1, acc = pl.pallas_call(
        _tc_comb_body,
        grid=grid,
        in_specs=[blk, blk, blk, pl.BlockSpec((k, d, d), lambda i: (0, 0, 0))],
        out_specs=[blk, blk],
        out_shape=[jax.ShapeDtypeStruct((n, d), jnp.float32),
                   jax.ShapeDtypeStruct((n, d), jnp.float32)],
    )(u, p0, p1, w)
    return t1, acc


def _tc_out_body(relu, acc_ref, u_ref, q0_ref, q1_ref, w2_ref, b_ref, o_ref):
    t2 = 2.0 * (q0_ref[...] + q1_ref[...]) - u_ref[...]
    o = (acc_ref[...]
         + jnp.dot(t2, w2_ref[...], preferred_element_type=jnp.float32)
         + b_ref[...])
    o_ref[...] = jnp.maximum(o, 0.0) if relu else o


def _tc_out(acc, u, q0, q1, w2, b, relu):
    n, d = u.shape
    blk = pl.BlockSpec((BM, d), lambda i: (i, 0))
    return pl.pallas_call(
        functools.partial(_tc_out_body, relu),
        grid=(n // BM,),
        in_specs=[blk, blk, blk, blk,
                  pl.BlockSpec((d, d), lambda i: (0, 0)),
                  pl.BlockSpec((1, d), lambda i: (0, 0))],
        out_specs=blk,
        out_shape=jax.ShapeDtypeStruct((n, d), jnp.float32),
    )(acc, u, q0, q1, w2, b.reshape(1, d))


def kernel(x, edge_index, edge_weight, W1, b1, W2, b2):
    n, d = x.shape
    row = edge_index[0]
    col = edge_index[1]
    npad = ((n + 1023) // 1024) * 1024

    degp = _sc_deg(row, col, edge_weight, npad)
    dinv = _tc_dinv(degp, npad)
    nrm = _sc_norm(row, col, edge_weight, dinv, npad)

    e = row.shape[0]
    epad = -e % (NW * BEDGE)
    rowp = jnp.concatenate([row, jnp.zeros((epad,), jnp.int32)])
    colp = jnp.concatenate([col, jnp.zeros((epad,), jnp.int32)])
    nrmp = jnp.concatenate([nrm, jnp.zeros((epad,), jnp.float32)])

    h = x
    for w, b, relu in ((W1, b1, True), (W2, b2, False)):
        p = _sc_prop(h, rowp, colp, nrmp)
        t1, acc = _tc_comb(h, p[:n], p[n:], w)
        q = _sc_prop(t1, rowp, colp, nrmp)
        h = _tc_out(acc, h, q[:n], q[n:], w[2], b, relu)
    return h


# speedup vs baseline: 1.8734x; 1.8734x over previous
import functools

import jax
import jax.numpy as jnp
from jax import lax
from jax.experimental import pallas as pl
from jax.experimental.pallas import tpu as pltpu
from jax.experimental.pallas import tpu_sc as plsc

NC = 2
NS = 16
NW = NC * NS
L = 16
BEDGE = 80
BM = 1000


def _mesh():
    return plsc.VectorSubcoreMesh(core_axis_name="c", subcore_axis_name="s")


_SC_PARAMS = pltpu.CompilerParams(needs_layout_passes=False)


def _sc_deg_body(ech, npad, row_h, col_h, w_h, z_h, out_h,
                 row_v, col_v, w_v, deg_v):
    wid = lax.axis_index("s") * NC + lax.axis_index("c")
    pltpu.sync_copy(z_h, deg_v)
    off = wid * ech
    pltpu.sync_copy(row_h.at[pl.ds(off, ech)], row_v)
    pltpu.sync_copy(col_h.at[pl.ds(off, ech)], col_v)
    pltpu.sync_copy(w_h.at[pl.ds(off, ech)], w_v)

    def body(i, carry):
        sl = pl.ds(i * L, L)
        r = row_v[sl]
        c = col_v[sl]
        w = w_v[sl]
        wz = jnp.where(r == c, 0.0, w)
        plsc.addupdate_scatter(deg_v, [r], wz)
        return carry

    lax.fori_loop(0, ech // L, body, 0)
    pltpu.sync_copy(deg_v, out_h.at[wid])


def _sc_deg(row, col, w, npad):
    e = row.shape[0]
    ech = e // NW
    z = jnp.zeros((npad,), jnp.float32)
    fn = pl.kernel(
        functools.partial(_sc_deg_body, ech, npad),
        out_type=jax.ShapeDtypeStruct((NW, npad), jnp.float32),
        mesh=_mesh(),
        compiler_params=_SC_PARAMS,
        scratch_types=[
            pltpu.VMEM((ech,), jnp.int32),
            pltpu.VMEM((ech,), jnp.int32),
            pltpu.VMEM((ech,), jnp.float32),
            pltpu.VMEM((npad,), jnp.float32),
        ],
    )
    return fn(row, col, w, z)


def _tc_dinv_body(d_ref, o_ref):
    d = jnp.sum(d_ref[...], axis=0)
    safe = jnp.where(d > 0.0, d, 1.0)
    o_ref[...] = jnp.where(d > 0.0, lax.rsqrt(safe), 0.0)


def _tc_dinv(degp, npad):
    rows = npad // 128
    degp3 = degp.reshape(NW, rows, 128)
    out = pl.pallas_call(
        _tc_dinv_body,
        grid=(rows // 8,),
        in_specs=[pl.BlockSpec((NW, 8, 128), lambda i: (0, i, 0))],
        out_specs=pl.BlockSpec((8, 128), lambda i: (i, 0)),
        out_shape=jax.ShapeDtypeStruct((rows, 128), jnp.float32),
    )(degp3)
    return out.reshape(npad)


def _sc_norm_body(ech, npad, row_h, col_h, w_h, dinv_h, out_h,
                  row_v, col_v, w_v, dv, nrm_v):
    wid = lax.axis_index("s") * NC + lax.axis_index("c")
    off = wid * ech
    pltpu.sync_copy(dinv_h, dv)
    pltpu.sync_copy(row_h.at[pl.ds(off, ech)], row_v)
    pltpu.sync_copy(col_h.at[pl.ds(off, ech)], col_v)
    pltpu.sync_copy(w_h.at[pl.ds(off, ech)], w_v)

    def body(i, carry):
        sl = pl.ds(i * L, L)
        r = row_v[sl]
        c = col_v[sl]
        w = w_v[sl]
        dr = plsc.load_gather(dv, [r])
        dc = plsc.load_gather(dv, [c])
        wz = jnp.where(r == c, 0.0, w)
        nrm_v[sl] = -(dr * wz * dc)
        return carry

    lax.fori_loop(0, ech // L, body, 0)
    pltpu.sync_copy(nrm_v, out_h.at[pl.ds(off, ech)])


def _sc_norm(row, col, w, dinv, npad):
    e = row.shape[0]
    ech = e // NW
    fn = pl.kernel(
        functools.partial(_sc_norm_body, ech, npad),
        out_type=jax.ShapeDtypeStruct((e,), jnp.float32),
        mesh=_mesh(),
        compiler_params=_SC_PARAMS,
        scratch_types=[
            pltpu.VMEM((ech,), jnp.int32),
            pltpu.VMEM((ech,), jnp.int32),
            pltpu.VMEM((ech,), jnp.float32),
            pltpu.VMEM((npad,), jnp.float32),
            pltpu.VMEM((ech,), jnp.float32),
        ],
    )
    return fn(row, col, w, dinv)


def _sc_prop_body(npad, ech, d, row_h, col_h, nrm_h, h_h, z_h, out_h,
                  rows_v, rowb0, rowb1, rowb2, rowb3,
                  colb0, colb1, colb2, colb3,
                  nrmb0, nrmb1, nrmb2, nrmb3,
                  sg0, sg1, sg2, sg3, ss0, ss1, ss2, ss3,
                  scr0, scr1, scr2, scr3, scc0, scc1, scc2, scc3,
                  scn0, scn1, scn2, scn3, acc_sp):
    cid = lax.axis_index("c")
    sid = lax.axis_index("s")
    wid = sid * NC + cid
    rpt = npad // NS
    nblk = ech // BEDGE
    rowb = (rowb0, rowb1, rowb2, rowb3)
    colb = (colb0, colb1, colb2, colb3)
    nrmb = (nrmb0, nrmb1, nrmb2, nrmb3)
    sg = (sg0, sg1, sg2, sg3)
    ss = (ss0, ss1, ss2, ss3)
    scr = (scr0, scr1, scr2, scr3)
    scc = (scc0, scc1, scc2, scc3)
    scn = (scn0, scn1, scn2, scn3)
    ebase = wid * ech

    def idx_dma(i, s4):
        off = ebase + i * BEDGE
        pltpu.async_copy(row_h.at[pl.ds(off, BEDGE)], rowb[s4], scr[s4])
        pltpu.async_copy(col_h.at[pl.ds(off, BEDGE)], colb[s4], scc[s4])
        pltpu.async_copy(nrm_h.at[pl.ds(off, BEDGE)], nrmb[s4], scn[s4])

    def idx_wait(s4):
        pltpu.make_async_copy(row_h.at[pl.ds(0, BEDGE)], rowb[s4],
                              scr[s4]).wait()
        pltpu.make_async_copy(col_h.at[pl.ds(0, BEDGE)], colb[s4],
                              scc[s4]).wait()
        pltpu.make_async_copy(nrm_h.at[pl.ds(0, BEDGE)], nrmb[s4],
                              scn[s4]).wait()

    def gather(s4):
        pltpu.async_copy(h_h.at[rowb[s4]], rows_v.at[s4], sg[s4])

    pltpu.sync_copy(z_h, acc_sp.at[pl.ds(sid * rpt, rpt)])
    idx_dma(0, 0)
    idx_dma(1, 1)
    idx_wait(0)
    gather(0)
    idx_wait(1)
    gather(1)
    plsc.subcore_barrier()

    nsup = (nblk + 3) // 4

    def sup(s, carry):
        for b in range(4):
            pb = (b + 2) % 4
            i = s * 4 + b

            @pl.when(jnp.logical_and(i >= 2, i - 2 < nblk))
            def _():
                pltpu.make_async_copy(
                    rows_v.at[pb], acc_sp.at[colb[0]], ss[pb]).wait()

            @pl.when(i + 2 < nblk)
            def _():
                idx_dma(i + 2, pb)

            @pl.when(i < nblk)
            def _():
                pltpu.make_async_copy(h_h.at[rowb[b]], rows_v.at[b],
                                      sg[b]).wait()

                def edge(e2, c2):
                    s16 = plsc.load_gather(
                        nrmb[b], [jnp.zeros((L,), jnp.int32) + e2])
                    for j in range(d // L):
                        sl = pl.ds(j * L, L)
                        rows_v[b, e2, sl] = rows_v[b, e2, sl] * s16
                    return c2

                lax.fori_loop(0, BEDGE, edge, 0, unroll=8)
                pltpu.async_copy(rows_v.at[b], acc_sp.at[colb[b]], ss[b],
                                 add=True)

            @pl.when(i + 2 < nblk)
            def _():
                idx_wait(pb)
                gather(pb)
        return carry

    lax.fori_loop(0, nsup, sup, 0)
    for j in range(max(0, 4 * nsup - 2), nblk):
        pltpu.make_async_copy(rows_v.at[j % 4], acc_sp.at[colb[0]],
                              ss[j % 4]).wait()
    plsc.subcore_barrier()
    pltpu.sync_copy(acc_sp.at[pl.ds(sid * rpt, rpt)],
                    out_h.at[pl.ds(cid * npad + sid * rpt, rpt)])


def _sc_prop(h, row, col, nrm, npad):
    n, d = h.shape
    e = row.shape[0]
    ech = e // NW
    rpt = npad // NS
    z = jnp.zeros((rpt, d), jnp.float32)
    fn = pl.kernel(
        functools.partial(_sc_prop_body, npad, ech, d),
        out_type=jax.ShapeDtypeStruct((NC * npad, d), jnp.float32),
        mesh=_mesh(),
        compiler_params=_SC_PARAMS,
        scratch_types=[
            pltpu.VMEM((4, BEDGE, d), jnp.float32),
            pltpu.VMEM((BEDGE,), jnp.int32),
            pltpu.VMEM((BEDGE,), jnp.int32),
            pltpu.VMEM((BEDGE,), jnp.int32),
            pltpu.VMEM((BEDGE,), jnp.int32),
            pltpu.VMEM((BEDGE,), jnp.int32),
            pltpu.VMEM((BEDGE,), jnp.int32),
            pltpu.VMEM((BEDGE,), jnp.int32),
            pltpu.VMEM((BEDGE,), jnp.int32),
            pltpu.VMEM((BEDGE,), jnp.float32),
            pltpu.VMEM((BEDGE,), jnp.float32),
            pltpu.VMEM((BEDGE,), jnp.float32),
            pltpu.VMEM((BEDGE,), jnp.float32),
        ] + [pltpu.SemaphoreType.DMA] * 20 + [
            pltpu.VMEM_SHARED((npad, d), jnp.float32),
        ],
    )
    return fn(row, col, nrm, h, z)


def _tc_comb_body(u_ref, p0_ref, p1_ref, w_ref, t1_ref, acc_ref):
    t1 = p0_ref[...] + p1_ref[...]
    t1_ref[...] = t1
    acc_ref[...] = (
        jnp.dot(u_ref[...], w_ref[0], preferred_element_type=jnp.float32)
        + jnp.dot(t1, w_ref[1], preferred_element_type=jnp.float32))


def _tc_comb(u, p0, p1, w):
    n, d = u.shape
    k = w.shape[0]
    grid = (n // BM,)
    blk = pl.BlockSpec((BM, d), lambda i: (i, 0))
    t1, acc = pl.pallas_call(
        _tc_comb_body,
        grid=grid,
        in_specs=[blk, blk, blk, pl.BlockSpec((k, d, d), lambda i: (0, 0, 0))],
        out_specs=[blk, blk],
        out_shape=[jax.ShapeDtypeStruct((n, d), jnp.float32),
                   jax.ShapeDtypeStruct((n, d), jnp.float32)],
    )(u, p0, p1, w)
    return t1, acc


def _tc_out_body(relu, acc_ref, u_ref, q0_ref, q1_ref, w2_ref, b_ref, o_ref):
    t2 = 2.0 * (q0_ref[...] + q1_ref[...]) - u_ref[...]
    o = (acc_ref[...]
         + jnp.dot(t2, w2_ref[...], preferred_element_type=jnp.float32)
         + b_ref[...])
    o_ref[...] = jnp.maximum(o, 0.0) if relu else o


def _tc_out(acc, u, q0, q1, w2, b, relu):
    n, d = u.shape
    blk = pl.BlockSpec((BM, d), lambda i: (i, 0))
    return pl.pallas_call(
        functools.partial(_tc_out_body, relu),
        grid=(n // BM,),
        in_specs=[blk, blk, blk, blk,
                  pl.BlockSpec((d, d), lambda i: (0, 0)),
                  pl.BlockSpec((1, d), lambda i: (0, 0))],
        out_specs=blk,
        out_shape=jax.ShapeDtypeStruct((n, d), jnp.float32),
    )(acc, u, q0, q1, w2, b.reshape(1, d))


def kernel(x, edge_index, edge_weight, W1, b1, W2, b2):
    n, d = x.shape
    row = edge_index[0]
    col = edge_index[1]
    npad = ((n + 1023) // 1024) * 1024

    degp = _sc_deg(row, col, edge_weight, npad)
    dinv = _tc_dinv(degp, npad)
    nrm = _sc_norm(row, col, edge_weight, dinv, npad)

    h = x
    for w, b, relu in ((W1, b1, True), (W2, b2, False)):
        p = _sc_prop(h, row, col, nrm, npad)
        t1, acc = _tc_comb(h, p[:n], p[npad:npad + n], w)
        q = _sc_prop(t1, row, col, nrm, npad)
        h = _tc_out(acc, h, q[:n], q[npad:npad + n], w[2], b, relu)
    return h


# speedup vs baseline: 1.8776x; 1.0022x over previous
import functools

import jax
import jax.numpy as jnp
from jax import lax
from jax.experimental import pallas as pl
from jax.experimental.pallas import tpu as pltpu
from jax.experimental.pallas import tpu_sc as plsc

NC = 2
NS = 16
NW = NC * NS
L = 16
BEDGE = 80
BM = 1000


def _mesh():
    return plsc.VectorSubcoreMesh(core_axis_name="c", subcore_axis_name="s")


_SC_PARAMS = pltpu.CompilerParams(needs_layout_passes=False)


def _sc_deg_body(ech, npad, row_h, col_h, w_h, z_h, out_h,
                 row_v, col_v, w_v, deg_v):
    wid = lax.axis_index("s") * NC + lax.axis_index("c")
    pltpu.sync_copy(z_h, deg_v)
    off = wid * ech
    pltpu.sync_copy(row_h.at[pl.ds(off, ech)], row_v)
    pltpu.sync_copy(col_h.at[pl.ds(off, ech)], col_v)
    pltpu.sync_copy(w_h.at[pl.ds(off, ech)], w_v)

    def body(i, carry):
        sl = pl.ds(i * L, L)
        r = row_v[sl]
        c = col_v[sl]
        w = w_v[sl]
        wz = jnp.where(r == c, 0.0, w)
        plsc.addupdate_scatter(deg_v, [r], wz)
        return carry

    lax.fori_loop(0, ech // L, body, 0)
    pltpu.sync_copy(deg_v, out_h.at[wid])


def _sc_deg(row, col, w, npad):
    e = row.shape[0]
    ech = e // NW
    z = jnp.zeros((npad,), jnp.float32)
    fn = pl.kernel(
        functools.partial(_sc_deg_body, ech, npad),
        out_type=jax.ShapeDtypeStruct((NW, npad), jnp.float32),
        mesh=_mesh(),
        compiler_params=_SC_PARAMS,
        scratch_types=[
            pltpu.VMEM((ech,), jnp.int32),
            pltpu.VMEM((ech,), jnp.int32),
            pltpu.VMEM((ech,), jnp.float32),
            pltpu.VMEM((npad,), jnp.float32),
        ],
    )
    return fn(row, col, w, z)


def _tc_dinv_body(d_ref, o_ref):
    d = jnp.sum(d_ref[...], axis=0)
    safe = jnp.where(d > 0.0, d, 1.0)
    o_ref[...] = jnp.where(d > 0.0, lax.rsqrt(safe), 0.0)


def _tc_dinv(degp, npad):
    rows = npad // 128
    degp3 = degp.reshape(NW, rows, 128)
    out = pl.pallas_call(
        _tc_dinv_body,
        grid=(rows // 8,),
        in_specs=[pl.BlockSpec((NW, 8, 128), lambda i: (0, i, 0))],
        out_specs=pl.BlockSpec((8, 128), lambda i: (i, 0)),
        out_shape=jax.ShapeDtypeStruct((rows, 128), jnp.float32),
    )(degp3)
    return out.reshape(npad)


def _sc_norm_body(ech, npad, row_h, col_h, w_h, dinv_h, out_h,
                  row_v, col_v, w_v, dv, nrm_v):
    wid = lax.axis_index("s") * NC + lax.axis_index("c")
    off = wid * ech
    pltpu.sync_copy(dinv_h, dv)
    pltpu.sync_copy(row_h.at[pl.ds(off, ech)], row_v)
    pltpu.sync_copy(col_h.at[pl.ds(off, ech)], col_v)
    pltpu.sync_copy(w_h.at[pl.ds(off, ech)], w_v)

    def body(i, carry):
        sl = pl.ds(i * L, L)
        r = row_v[sl]
        c = col_v[sl]
        w = w_v[sl]
        dr = plsc.load_gather(dv, [r])
        dc = plsc.load_gather(dv, [c])
        wz = jnp.where(r == c, 0.0, w)
        nrm_v[sl] = -(dr * wz * dc)
        return carry

    lax.fori_loop(0, ech // L, body, 0)
    pltpu.sync_copy(nrm_v, out_h.at[pl.ds(off, ech)])


def _sc_norm(row, col, w, dinv, npad):
    e = row.shape[0]
    ech = e // NW
    fn = pl.kernel(
        functools.partial(_sc_norm_body, ech, npad),
        out_type=jax.ShapeDtypeStruct((e,), jnp.float32),
        mesh=_mesh(),
        compiler_params=_SC_PARAMS,
        scratch_types=[
            pltpu.VMEM((ech,), jnp.int32),
            pltpu.VMEM((ech,), jnp.int32),
            pltpu.VMEM((ech,), jnp.float32),
            pltpu.VMEM((npad,), jnp.float32),
            pltpu.VMEM((ech,), jnp.float32),
        ],
    )
    return fn(row, col, w, dinv)


def _sc_prop_body(npad, ech, d, row_h, col_h, nrm_h, h_h, z_h, out_h,
                  rows_v, rowb0, rowb1, rowb2, rowb3,
                  colb0, colb1, colb2, colb3,
                  nrmb0, nrmb1, nrmb2, nrmb3,
                  sg0, sg1, sg2, sg3, ss0, ss1, ss2, ss3,
                  scr0, scr1, scr2, scr3, scc0, scc1, scc2, scc3,
                  scn0, scn1, scn2, scn3, acc_sp):
    cid = lax.axis_index("c")
    sid = lax.axis_index("s")
    wid = sid * NC + cid
    rpt = npad // NS
    nblk = ech // BEDGE
    rowb = (rowb0, rowb1, rowb2, rowb3)
    colb = (colb0, colb1, colb2, colb3)
    nrmb = (nrmb0, nrmb1, nrmb2, nrmb3)
    sg = (sg0, sg1, sg2, sg3)
    ss = (ss0, ss1, ss2, ss3)
    scr = (scr0, scr1, scr2, scr3)
    scc = (scc0, scc1, scc2, scc3)
    scn = (scn0, scn1, scn2, scn3)
    ebase = wid * ech

    def idx_dma(i, s4):
        off = ebase + i * BEDGE
        pltpu.async_copy(row_h.at[pl.ds(off, BEDGE)], rowb[s4], scr[s4])
        pltpu.async_copy(col_h.at[pl.ds(off, BEDGE)], colb[s4], scc[s4])
        pltpu.async_copy(nrm_h.at[pl.ds(off, BEDGE)], nrmb[s4], scn[s4])

    def idx_wait(s4):
        pltpu.make_async_copy(row_h.at[pl.ds(0, BEDGE)], rowb[s4],
                              scr[s4]).wait()
        pltpu.make_async_copy(col_h.at[pl.ds(0, BEDGE)], colb[s4],
                              scc[s4]).wait()
        pltpu.make_async_copy(nrm_h.at[pl.ds(0, BEDGE)], nrmb[s4],
                              scn[s4]).wait()

    def gather(s4):
        pltpu.async_copy(h_h.at[rowb[s4]], rows_v.at[s4], sg[s4])

    pltpu.sync_copy(z_h, acc_sp.at[pl.ds(sid * rpt, rpt)])
    idx_dma(0, 0)
    idx_dma(1, 1)
    idx_wait(0)
    gather(0)
    idx_wait(1)
    gather(1)
    plsc.subcore_barrier()

    nsup = (nblk + 3) // 4

    def sup(s, carry):
        for b in range(4):
            pb = (b + 2) % 4
            i = s * 4 + b

            @pl.when(jnp.logical_and(i >= 2, i - 2 < nblk))
            def _():
                pltpu.make_async_copy(
                    rows_v.at[pb], acc_sp.at[colb[0]], ss[pb]).wait()

            @pl.when(i + 2 < nblk)
            def _():
                idx_dma(i + 2, pb)

            @pl.when(i < nblk)
            def _():
                pltpu.make_async_copy(h_h.at[rowb[b]], rows_v.at[b],
                                      sg[b]).wait()

                def edge(e2, c2):
                    s16 = plsc.load_gather(
                        nrmb[b], [jnp.zeros((L,), jnp.int32) + e2])
                    for j in range(d // L):
                        sl = pl.ds(j * L, L)
                        rows_v[b, e2, sl] = rows_v[b, e2, sl] * s16
                    return c2

                lax.fori_loop(0, BEDGE, edge, 0, unroll=8)
                pltpu.async_copy(rows_v.at[b], acc_sp.at[colb[b]], ss[b],
                                 add=True)

            @pl.when(i + 2 < nblk)
            def _():
                idx_wait(pb)
                gather(pb)
        return carry

    lax.fori_loop(0, nsup, sup, 0)
    for j in range(max(0, 4 * nsup - 2), nblk):
        pltpu.make_async_copy(rows_v.at[j % 4], acc_sp.at[colb[0]],
                              ss[j % 4]).wait()
    plsc.subcore_barrier()
    pltpu.sync_copy(acc_sp.at[pl.ds(sid * rpt, rpt)],
                    out_h.at[pl.ds(cid * npad + sid * rpt, rpt)])


def _sc_prop(h, row, col, nrm, npad):
    n, d = h.shape
    e = row.shape[0]
    ech = e // NW
    rpt = npad // NS
    z = jnp.zeros((rpt, d), jnp.float32)
    fn = pl.kernel(
        functools.partial(_sc_prop_body, npad, ech, d),
        out_type=jax.ShapeDtypeStruct((NC * npad, d), jnp.float32),
        mesh=_mesh(),
        compiler_params=_SC_PARAMS,
        scratch_types=[
            pltpu.VMEM((4, BEDGE, d), jnp.float32),
            pltpu.VMEM((BEDGE,), jnp.int32),
            pltpu.VMEM((BEDGE,), jnp.int32),
            pltpu.VMEM((BEDGE,), jnp.int32),
            pltpu.VMEM((BEDGE,), jnp.int32),
            pltpu.VMEM((BEDGE,), jnp.int32),
            pltpu.VMEM((BEDGE,), jnp.int32),
            pltpu.VMEM((BEDGE,), jnp.int32),
            pltpu.VMEM((BEDGE,), jnp.int32),
            pltpu.VMEM((BEDGE,), jnp.float32),
            pltpu.VMEM((BEDGE,), jnp.float32),
            pltpu.VMEM((BEDGE,), jnp.float32),
            pltpu.VMEM((BEDGE,), jnp.float32),
        ] + [pltpu.SemaphoreType.DMA] * 20 + [
            pltpu.VMEM_SHARED((npad, d), jnp.float32),
        ],
    )
    return fn(row, col, nrm, h, z)


def _tc_t1_body(p0_ref, p1_ref, t1_ref):
    t1_ref[...] = p0_ref[...] + p1_ref[...]


def _tc_t1(p0, p1):
    n, d = p0.shape
    blk = pl.BlockSpec((BM, d), lambda i: (i, 0))
    return pl.pallas_call(
        _tc_t1_body,
        grid=(n // BM,),
        in_specs=[blk, blk],
        out_specs=blk,
        out_shape=jax.ShapeDtypeStruct((n, d), jnp.float32),
    )(p0, p1)


def _tc_acc_body(u_ref, t1_ref, w_ref, acc_ref):
    acc_ref[...] = (
        jnp.dot(u_ref[...], w_ref[0], preferred_element_type=jnp.float32)
        + jnp.dot(t1_ref[...], w_ref[1], preferred_element_type=jnp.float32))


def _tc_acc(u, t1, w):
    n, d = u.shape
    k = w.shape[0]
    blk = pl.BlockSpec((BM, d), lambda i: (i, 0))
    return pl.pallas_call(
        _tc_acc_body,
        grid=(n // BM,),
        in_specs=[blk, blk, pl.BlockSpec((k, d, d), lambda i: (0, 0, 0))],
        out_specs=blk,
        out_shape=jax.ShapeDtypeStruct((n, d), jnp.float32),
    )(u, t1, w)


def _tc_out_body(relu, acc_ref, u_ref, q0_ref, q1_ref, w2_ref, b_ref, o_ref):
    t2 = 2.0 * (q0_ref[...] + q1_ref[...]) - u_ref[...]
    o = (acc_ref[...]
         + jnp.dot(t2, w2_ref[...], preferred_element_type=jnp.float32)
         + b_ref[...])
    o_ref[...] = jnp.maximum(o, 0.0) if relu else o


def _tc_out(acc, u, q0, q1, w2, b, relu):
    n, d = u.shape
    blk = pl.BlockSpec((BM, d), lambda i: (i, 0))
    return pl.pallas_call(
        functools.partial(_tc_out_body, relu),
        grid=(n // BM,),
        in_specs=[blk, blk, blk, blk,
                  pl.BlockSpec((d, d), lambda i: (0, 0)),
                  pl.BlockSpec((1, d), lambda i: (0, 0))],
        out_specs=blk,
        out_shape=jax.ShapeDtypeStruct((n, d), jnp.float32),
    )(acc, u, q0, q1, w2, b.reshape(1, d))


def kernel(x, edge_index, edge_weight, W1, b1, W2, b2):
    n, d = x.shape
    row = edge_index[0]
    col = edge_index[1]
    npad = ((n + 1023) // 1024) * 1024

    degp = _sc_deg(row, col, edge_weight, npad)
    dinv = _tc_dinv(degp, npad)
    nrm = _sc_norm(row, col, edge_weight, dinv, npad)

    h = x
    for w, b, relu in ((W1, b1, True), (W2, b2, False)):
        p = _sc_prop(h, row, col, nrm, npad)
        t1 = _tc_t1(p[:n], p[npad:npad + n])
        q = _sc_prop(t1, row, col, nrm, npad)
        acc = _tc_acc(h, t1, w)
        h = _tc_out(acc, h, q[:n], q[npad:npad + n], w[2], b, relu)
    return h


# speedup vs baseline: 1.8838x; 1.0033x over previous
import functools

import jax
import jax.numpy as jnp
from jax import lax
from jax.experimental import pallas as pl
from jax.experimental.pallas import tpu as pltpu
from jax.experimental.pallas import tpu_sc as plsc

NC = 2
NS = 16
NW = NC * NS
L = 16
BEDGE = 80
BM = 1000


def _mesh():
    return plsc.VectorSubcoreMesh(core_axis_name="c", subcore_axis_name="s")


_SC_PARAMS = pltpu.CompilerParams(needs_layout_passes=False)


def _sc_prep_body(e, npad, row_h, col_h, w_h, z_h, out_h,
                  row_v, col_v, w_v, deg_v, dbuf, dvloc, dv, nrm_v,
                  deg_sh, dv_sh):
    cid = lax.axis_index("c")
    sid = lax.axis_index("s")
    etile = e // NS
    nslc = npad // NS

    pltpu.sync_copy(z_h, deg_v)
    off = sid * etile
    pltpu.sync_copy(row_h.at[pl.ds(off, etile)], row_v)
    pltpu.sync_copy(col_h.at[pl.ds(off, etile)], col_v)
    pltpu.sync_copy(w_h.at[pl.ds(off, etile)], w_v)

    def body(i, carry):
        sl = pl.ds(i * L, L)
        r = row_v[sl]
        c = col_v[sl]
        w = w_v[sl]
        wz = jnp.where(r == c, 0.0, w)
        plsc.addupdate_scatter(deg_v, [r], wz)
        return carry

    lax.fori_loop(0, etile // L, body, 0, unroll=4)
    pltpu.sync_copy(deg_v, deg_sh.at[sid])
    plsc.subcore_barrier()

    for k in range(NS):
        pltpu.sync_copy(deg_sh.at[k, pl.ds(sid * nslc, nslc)], dbuf.at[k])
    magic = jnp.full((L,), 0x5f3759df, jnp.int32)

    def newton(i, carry):
        sl = pl.ds(i * L, L)
        d = dbuf[0, sl]
        for k in range(1, NS):
            d = d + dbuf[k, sl]
        y = plsc.bitcast(magic - lax.shift_right_logical(
            plsc.bitcast(d, jnp.int32), 1), jnp.float32)
        for _ in range(4):
            y = y * (1.5 - 0.5 * d * y * y)
        dvloc[sl] = jnp.where(d > 0.0, y, 0.0)
        return carry

    lax.fori_loop(0, nslc // L, newton, 0, unroll=2)
    pltpu.sync_copy(dvloc, dv_sh.at[pl.ds(sid * nslc, nslc)])
    plsc.subcore_barrier()

    pltpu.sync_copy(dv_sh, dv)
    half = etile // NC
    loc = cid * half

    def body3(i, carry):
        sl = pl.ds(loc + i * L, L)
        r = row_v[sl]
        c = col_v[sl]
        w = w_v[sl]
        dr = plsc.load_gather(dv, [r])
        dc = plsc.load_gather(dv, [c])
        wz = jnp.where(r == c, 0.0, w)
        nrm_v[pl.ds(i * L, L)] = -(dr * wz * dc)
        return carry

    lax.fori_loop(0, half // L, body3, 0, unroll=4)
    pltpu.sync_copy(nrm_v, out_h.at[pl.ds(off + loc, half)])


def _sc_prep(row, col, w, npad):
    e = row.shape[0]
    etile = e // NS
    z = jnp.zeros((npad,), jnp.float32)
    fn = pl.kernel(
        functools.partial(_sc_prep_body, e, npad),
        out_type=jax.ShapeDtypeStruct((e,), jnp.float32),
        mesh=_mesh(),
        compiler_params=_SC_PARAMS,
        scratch_types=[
            pltpu.VMEM((etile,), jnp.int32),
            pltpu.VMEM((etile,), jnp.int32),
            pltpu.VMEM((etile,), jnp.float32),
            pltpu.VMEM((npad,), jnp.float32),
            pltpu.VMEM((NS, npad // NS), jnp.float32),
            pltpu.VMEM((npad // NS,), jnp.float32),
            pltpu.VMEM((npad,), jnp.float32),
            pltpu.VMEM((etile // NC,), jnp.float32),
            pltpu.VMEM_SHARED((NS, npad), jnp.float32),
            pltpu.VMEM_SHARED((npad,), jnp.float32),
        ],
    )
    return fn(row, col, w, z)


def _sc_prop_body(npad, ech, d, row_h, col_h, nrm_h, h_h, z_h, out_h,
                  rows_v, rowb0, rowb1, rowb2, rowb3,
                  colb0, colb1, colb2, colb3,
                  nrmb0, nrmb1, nrmb2, nrmb3,
                  sg0, sg1, sg2, sg3, ss0, ss1, ss2, ss3,
                  scr0, scr1, scr2, scr3, scc0, scc1, scc2, scc3,
                  scn0, scn1, scn2, scn3, acc_sp):
    cid = lax.axis_index("c")
    sid = lax.axis_index("s")
    wid = sid * NC + cid
    rpt = npad // NS
    nblk = ech // BEDGE
    rowb = (rowb0, rowb1, rowb2, rowb3)
    colb = (colb0, colb1, colb2, colb3)
    nrmb = (nrmb0, nrmb1, nrmb2, nrmb3)
    sg = (sg0, sg1, sg2, sg3)
    ss = (ss0, ss1, ss2, ss3)
    scr = (scr0, scr1, scr2, scr3)
    scc = (scc0, scc1, scc2, scc3)
    scn = (scn0, scn1, scn2, scn3)
    ebase = wid * ech

    def idx_dma(i, s4):
        off = ebase + i * BEDGE
        pltpu.async_copy(row_h.at[pl.ds(off, BEDGE)], rowb[s4], scr[s4])
        pltpu.async_copy(col_h.at[pl.ds(off, BEDGE)], colb[s4], scc[s4])
        pltpu.async_copy(nrm_h.at[pl.ds(off, BEDGE)], nrmb[s4], scn[s4])

    def idx_wait(s4):
        pltpu.make_async_copy(row_h.at[pl.ds(0, BEDGE)], rowb[s4],
                              scr[s4]).wait()
        pltpu.make_async_copy(col_h.at[pl.ds(0, BEDGE)], colb[s4],
                              scc[s4]).wait()
        pltpu.make_async_copy(nrm_h.at[pl.ds(0, BEDGE)], nrmb[s4],
                              scn[s4]).wait()

    def gather(s4):
        pltpu.async_copy(h_h.at[rowb[s4]], rows_v.at[s4], sg[s4])

    pltpu.sync_copy(z_h, acc_sp.at[pl.ds(sid * rpt, rpt)])
    idx_dma(0, 0)
    idx_dma(1, 1)
    idx_wait(0)
    gather(0)
    idx_wait(1)
    gather(1)
    plsc.subcore_barrier()

    nsup = (nblk + 3) // 4

    def sup(s, carry):
        for b in range(4):
            pb = (b + 2) % 4
            i = s * 4 + b

            @pl.when(jnp.logical_and(i >= 2, i - 2 < nblk))
            def _():
                pltpu.make_async_copy(
                    rows_v.at[pb], acc_sp.at[colb[0]], ss[pb]).wait()

            @pl.when(i + 2 < nblk)
            def _():
                idx_dma(i + 2, pb)

            @pl.when(i < nblk)
            def _():
                pltpu.make_async_copy(h_h.at[rowb[b]], rows_v.at[b],
                                      sg[b]).wait()

                def edge(e2, c2):
                    s16 = plsc.load_gather(
                        nrmb[b], [jnp.zeros((L,), jnp.int32) + e2])
                    for j in range(d // L):
                        sl = pl.ds(j * L, L)
                        rows_v[b, e2, sl] = rows_v[b, e2, sl] * s16
                    return c2

                lax.fori_loop(0, BEDGE, edge, 0, unroll=8)
                pltpu.async_copy(rows_v.at[b], acc_sp.at[colb[b]], ss[b],
                                 add=True)

            @pl.when(i + 2 < nblk)
            def _():
                idx_wait(pb)
                gather(pb)
        return carry

    lax.fori_loop(0, nsup, sup, 0)
    for j in range(max(0, 4 * nsup - 2), nblk):
        pltpu.make_async_copy(rows_v.at[j % 4], acc_sp.at[colb[0]],
                              ss[j % 4]).wait()
    plsc.subcore_barrier()
    pltpu.sync_copy(acc_sp.at[pl.ds(sid * rpt, rpt)],
                    out_h.at[pl.ds(cid * npad + sid * rpt, rpt)])


def _sc_prop(h, row, col, nrm, npad):
    n, d = h.shape
    e = row.shape[0]
    ech = e // NW
    rpt = npad // NS
    z = jnp.zeros((rpt, d), jnp.float32)
    fn = pl.kernel(
        functools.partial(_sc_prop_body, npad, ech, d),
        out_type=jax.ShapeDtypeStruct((NC * npad, d), jnp.float32),
        mesh=_mesh(),
        compiler_params=_SC_PARAMS,
        scratch_types=[
            pltpu.VMEM((4, BEDGE, d), jnp.float32),
            pltpu.VMEM((BEDGE,), jnp.int32),
            pltpu.VMEM((BEDGE,), jnp.int32),
            pltpu.VMEM((BEDGE,), jnp.int32),
            pltpu.VMEM((BEDGE,), jnp.int32),
            pltpu.VMEM((BEDGE,), jnp.int32),
            pltpu.VMEM((BEDGE,), jnp.int32),
            pltpu.VMEM((BEDGE,), jnp.int32),
            pltpu.VMEM((BEDGE,), jnp.int32),
            pltpu.VMEM((BEDGE,), jnp.float32),
            pltpu.VMEM((BEDGE,), jnp.float32),
            pltpu.VMEM((BEDGE,), jnp.float32),
            pltpu.VMEM((BEDGE,), jnp.float32),
        ] + [pltpu.SemaphoreType.DMA] * 20 + [
            pltpu.VMEM_SHARED((npad, d), jnp.float32),
        ],
    )
    return fn(row, col, nrm, h, z)


def _tc_t1_body(p0_ref, p1_ref, t1_ref):
    t1_ref[...] = p0_ref[...] + p1_ref[...]


def _tc_t1(p0, p1):
    n, d = p0.shape
    blk = pl.BlockSpec((BM, d), lambda i: (i, 0))
    return pl.pallas_call(
        _tc_t1_body,
        grid=(n // BM,),
        in_specs=[blk, blk],
        out_specs=blk,
        out_shape=jax.ShapeDtypeStruct((n, d), jnp.float32),
    )(p0, p1)


def _tc_acc_body(u_ref, t1_ref, w_ref, acc_ref):
    acc_ref[...] = (
        jnp.dot(u_ref[...], w_ref[0], preferred_element_type=jnp.float32)
        + jnp.dot(t1_ref[...], w_ref[1], preferred_element_type=jnp.float32))


def _tc_acc(u, t1, w):
    n, d = u.shape
    k = w.shape[0]
    blk = pl.BlockSpec((BM, d), lambda i: (i, 0))
    return pl.pallas_call(
        _tc_acc_body,
        grid=(n // BM,),
        in_specs=[blk, blk, pl.BlockSpec((k, d, d), lambda i: (0, 0, 0))],
        out_specs=blk,
        out_shape=jax.ShapeDtypeStruct((n, d), jnp.float32),
    )(u, t1, w)


def _tc_out_body(relu, acc_ref, u_ref, q0_ref, q1_ref, w2_ref, b_ref, o_ref):
    t2 = 2.0 * (q0_ref[...] + q1_ref[...]) - u_ref[...]
    o = (acc_ref[...]
         + jnp.dot(t2, w2_ref[...], preferred_element_type=jnp.float32)
         + b_ref[...])
    o_ref[...] = jnp.maximum(o, 0.0) if relu else o


def _tc_out(acc, u, q0, q1, w2, b, relu):
    n, d = u.shape
    blk = pl.BlockSpec((BM, d), lambda i: (i, 0))
    return pl.pallas_call(
        functools.partial(_tc_out_body, relu),
        grid=(n // BM,),
        in_specs=[blk, blk, blk, blk,
                  pl.BlockSpec((d, d), lambda i: (0, 0)),
                  pl.BlockSpec((1, d), lambda i: (0, 0))],
        out_specs=blk,
        out_shape=jax.ShapeDtypeStruct((n, d), jnp.float32),
    )(acc, u, q0, q1, w2, b.reshape(1, d))


def kernel(x, edge_index, edge_weight, W1, b1, W2, b2):
    n, d = x.shape
    row = edge_index[0]
    col = edge_index[1]
    npad = ((n + 1023) // 1024) * 1024

    nrm = _sc_prep(row, col, edge_weight, npad)

    h = x
    for w, b, relu in ((W1, b1, True), (W2, b2, False)):
        p = _sc_prop(h, row, col, nrm, npad)
        t1 = _tc_t1(p[:n], p[npad:npad + n])
        q = _sc_prop(t1, row, col, nrm, npad)
        acc = _tc_acc(h, t1, w)
        h = _tc_out(acc, h, q[:n], q[npad:npad + n], w[2], b, relu)
    return h
